# baseline (speedup 1.0000x reference)
import functools

import jax
import jax.numpy as jnp
from jax import lax
from jax.experimental import pallas as pl
from jax.experimental.pallas import tpu as pltpu
from jax.experimental.pallas import tpu_sc as plsc


def _sc_death_body(attr_hbm, out_hbm, bufA, bufB, dv, pm, semA, semB,
                   *, B, C, L, NP, PS, NC, GROUPS, GROUPS_SC):
    wid = lax.axis_index("s") * NC + lax.axis_index("c")
    b_w = wid // 4
    slot = wid % 4
    gstep = (GROUPS_SC + 3) // 4
    trips = jnp.where(slot < 3, gstep, GROUPS_SC - 3 * gstep)
    pairs = trips // 2
    base_group = b_w * GROUPS + (GROUPS - GROUPS_SC) + slot * gstep

    GW = 16 * L

    def start(g, buf, sem):
        src = attr_hbm.at[pl.ds((base_group + g) * GW, GW)]
        pltpu.make_async_copy(src, buf, sem).start()

    def wait(buf, sem):
        pltpu.make_async_copy(attr_hbm.at[pl.ds(0, GW)], buf, sem).wait()

    def ranks(buf):
        offs = list(range(0, PS - 16, 16)) + [PS - 16]

        def row_body(r, carry):
            ro = r * L
            for q in range(NP):
                vs = [buf[pl.ds(ro + q * PS + o, 16)] for o in offs]
                while len(vs) > 1:
                    nxt = [jnp.maximum(vs[i], vs[i + 1]) for i in range(0, len(vs) - 1, 2)]
                    if len(vs) % 2:
                        nxt.append(vs[-1])
                    vs = nxt
                pm[pl.ds(q * 256 + r * 16, 16)] = vs[0]
            return carry

        lax.fori_loop(0, 16, row_body, 0)

        lane16 = lax.iota(jnp.int32, 16) * 16
        pooled = []
        for q in range(NP):
            idxv = lane16 + (q * 256)
            gs = [plsc.load_gather(pm, [idxv + j]) for j in range(16)]
            while len(gs) > 1:
                nxt = [jnp.maximum(gs[i], gs[i + 1]) for i in range(0, len(gs) - 1, 2)]
                if len(gs) % 2:
                    nxt.append(gs[-1])
                gs = nxt
            pooled.append(gs[0])
        rs = []
        for P in range(NP):
            acc = jnp.zeros((16,), jnp.float32)
            for Q in range(NP):
                if Q == P:
                    continue
                if Q < P:
                    cond = pooled[Q] <= pooled[P]
                else:
                    cond = pooled[Q] < pooled[P]
                acc = acc + cond.astype(jnp.float32)
            rs.append(acc)
        return rs

    start(0, bufA, semA)

    def pair_body(i, rmin):
        g0 = 2 * i
        start(g0 + 1, bufB, semB)
        wait(bufA, semA)
        rA = ranks(bufA)

        @pl.when(g0 + 2 < trips)
        def _():
            start(g0 + 2, bufA, semA)

        wait(bufB, semB)
        rB = ranks(bufB)
        return tuple(
            jnp.minimum(jnp.minimum(rmin[q], rA[q]), rB[q]) for q in range(NP)
        )

    init = tuple(jnp.full((16,), float(NP - 1), jnp.float32) for _ in range(NP))
    rmin = lax.fori_loop(0, pairs, pair_body, init)
    for q in range(NP):
        dv[q, :] = rmin[q]

    @pl.when(trips % 2 == 1)
    def _():
        wait(bufA, semA)
        rT = ranks(bufA)
        for q in range(NP):
            dv[q, :] = jnp.minimum(dv[q, :], rT[q])

    pltpu.sync_copy(dv, out_hbm.at[b_w, slot])


def _tc_sums_kernel(x_ref, attr_ref, s_ref, dtc_ref, death_acc, *, NP, PS, KP, L, AR):
    p = pl.program_id(1)
    for j in range(KP):
        s_ref[0, pl.ds(p * KP + j, 1), :] = jnp.sum(
            x_ref[0, j * PS:(j + 1) * PS, :], axis=0, keepdims=True
        )

    a = attr_ref[0]
    pooled = jnp.concatenate(
        [jnp.max(a[:, q * PS:(q + 1) * PS], axis=1, keepdims=True) for q in range(NP)],
        axis=1,
    )
    pt = jnp.transpose(pooled)

    r = jnp.zeros((NP, AR), dtype=jnp.int32)
    sub = jax.lax.broadcasted_iota(jnp.int32, (NP, AR), 0)
    for q in range(NP):
        vq = pt[q:q + 1, :]
        cond = (vq < pt) | ((vq == pt) & (sub > q))
        r = r + cond.astype(jnp.int32)

    @pl.when(p == 0)
    def _():
        death_acc[...] = r

    @pl.when(p > 0)
    def _():
        death_acc[...] = jnp.minimum(death_acc[...], r)

    @pl.when(p == (NP // KP) - 1)
    def _():
        d = jnp.min(death_acc[...], axis=1, keepdims=True).astype(jnp.float32)
        dtc_ref[0] = jnp.broadcast_to(d, (NP, 16))


def _tc_combine_kernel(s_ref, dp_ref, dtc_ref, out_ref, *, NP, C, L):
    dp = dp_ref[0]
    dmin = jnp.minimum(dp[0:NP, :], dtc_ref[0])
    for w in range(1, 4):
        dmin = jnp.minimum(dmin, dp[w * NP:(w + 1) * NP, :])
    death = jnp.min(dmin, axis=1, keepdims=True)

    stage = jax.lax.broadcasted_iota(jnp.int32, (NP, NP), 1).astype(jnp.float32)
    alive = (death >= stage).astype(jnp.float32)
    m = jax.lax.dot_general(
        alive, s_ref[0], (((0,), (0,)), ((), ())),
        preferred_element_type=jnp.float32,
    )
    m = m * (1.0 / C)

    m = m - jnp.max(m, axis=1, keepdims=True)
    e = jnp.exp(m)
    o = e / jnp.sum(e, axis=1, keepdims=True)

    u = 1.0 / L
    numer = jnp.sum(o[1:, :], axis=0, keepdims=True) + 0.5 * u
    res = (0.5 + numer / o[0:1, :]) * (1.0 / (NP + 1))
    out_ref[0] = res


@jax.jit
def kernel(x, attr, mask):
    B, C, L = x.shape
    PS = int(0.1 * L)
    NP = L // PS
    KP = 5
    NC, NS = 2, 16
    GROUPS = C // 16
    C_SC = 64
    GROUPS_SC = C_SC // 16
    AR = (C - C_SC) // (NP // KP)

    sc_death = functools.partial(
        pl.kernel,
        out_type=jax.ShapeDtypeStruct((B, 4, NP, 16), jnp.float32),
        mesh=plsc.VectorSubcoreMesh(core_axis_name="c", subcore_axis_name="s"),
        scratch_types=[
            pltpu.VMEM((16 * L,), jnp.float32),
            pltpu.VMEM((16 * L,), jnp.float32),
            pltpu.VMEM((NP, 16), jnp.float32),
            pltpu.VMEM((NP * 256,), jnp.float32),
            pltpu.SemaphoreType.DMA,
            pltpu.SemaphoreType.DMA,
        ],
        compiler_params=pltpu.CompilerParams(needs_layout_passes=False),
    )(functools.partial(_sc_death_body, B=B, C=C, L=L, NP=NP, PS=PS, NC=NC,
                        GROUPS=GROUPS, GROUPS_SC=GROUPS_SC))
    dp = sc_death(attr.reshape(-1))

    s, dtc = pl.pallas_call(
        functools.partial(_tc_sums_kernel, NP=NP, PS=PS, KP=KP, L=L, AR=AR),
        grid=(B, NP // KP),
        in_specs=[
            pl.BlockSpec((1, KP * PS, L), lambda b, p: (b, p, 0)),
            pl.BlockSpec((1, AR, L), lambda b, p: (b, p, 0)),
        ],
        out_specs=[
            pl.BlockSpec((1, NP, L), lambda b, p: (b, 0, 0)),
            pl.BlockSpec((1, NP, 16), lambda b, p: (b, 0, 0)),
        ],
        out_shape=[
            jax.ShapeDtypeStruct((B, NP, L), jnp.float32),
            jax.ShapeDtypeStruct((B, NP, 16), jnp.float32),
        ],
        scratch_shapes=[pltpu.VMEM((NP, AR), jnp.int32)],
        compiler_params=pltpu.CompilerParams(
            dimension_semantics=("parallel", "arbitrary"),
        ),
    )(x, attr)

    out = pl.pallas_call(
        functools.partial(_tc_combine_kernel, NP=NP, C=C, L=L),
        grid=(B,),
        in_specs=[
            pl.BlockSpec((1, NP, L), lambda b: (b, 0, 0)),
            pl.BlockSpec((1, 4 * NP, 16), lambda b: (b, 0, 0)),
            pl.BlockSpec((1, NP, 16), lambda b: (b, 0, 0)),
        ],
        out_specs=pl.BlockSpec((1, 1, L), lambda b: (b, 0, 0)),
        out_shape=jax.ShapeDtypeStruct((B, 1, L), jnp.float32),
    )(s, dp.reshape(B, 4 * NP, 16), dtc)
    return out.reshape(B, L)


# speedup vs baseline: 3.0722x; 3.0722x over previous
import functools

import jax
import jax.numpy as jnp
import numpy as np
from jax.experimental import pallas as pl
from jax.experimental.pallas import tpu as pltpu


def _infidelity_kernel(x_ref, attr_ref, out_ref, s_acc, death_acc, *, NP, PS, KP, C, L):
    p = pl.program_id(1)
    RB = KP * PS

    a = attr_ref[0]
    pooled = jnp.concatenate(
        [jnp.max(a[:, q * PS:(q + 1) * PS], axis=1, keepdims=True) for q in range(NP)],
        axis=1,
    )
    pt = jnp.transpose(pooled)

    r = jnp.zeros((NP, RB), dtype=jnp.int32)
    sub = jax.lax.broadcasted_iota(jnp.int32, (NP, RB), 0)
    for q in range(NP):
        vq = pt[q:q + 1, :]
        cond = (vq < pt) | ((vq == pt) & (sub > q))
        r = r + cond.astype(jnp.int32)

    @pl.when(p == 0)
    def _():
        death_acc[...] = r

    @pl.when(p > 0)
    def _():
        death_acc[...] = jnp.minimum(death_acc[...], r)

    for j in range(KP):
        s_acc[pl.ds(p * KP + j, 1), :] = jnp.sum(
            x_ref[0, j * PS:(j + 1) * PS, :], axis=0, keepdims=True
        )

    @pl.when(p == (NP // KP) - 1)
    def _():
        death = jnp.min(death_acc[...], axis=1, keepdims=True)
        stage = jax.lax.broadcasted_iota(jnp.int32, (NP, NP), 1)
        alive = (death >= stage).astype(jnp.float32)
        m = jax.lax.dot_general(
            alive, s_acc[...], (((0,), (0,)), ((), ())),
            preferred_element_type=jnp.float32,
        )
        m = m * (1.0 / C)

        m = m - jnp.max(m, axis=1, keepdims=True)
        e = jnp.exp(m)
        o = e / jnp.sum(e, axis=1, keepdims=True)

        u = 1.0 / L
        numer = jnp.sum(o[1:, :], axis=0, keepdims=True) + 0.5 * u
        res = (0.5 + numer / o[0:1, :]) * (1.0 / (NP + 1))
        out_ref[0] = res


@jax.jit
def kernel(x, attr, mask):
    B, C, L = x.shape
    PS = int(0.1 * L)
    NP = L // PS
    KP = 5

    grid = (B, NP // KP)
    out = pl.pallas_call(
        functools.partial(_infidelity_kernel, NP=NP, PS=PS, KP=KP, C=C, L=L),
        grid=grid,
        in_specs=[
            pl.BlockSpec((1, KP * PS, L), lambda b, p: (b, p, 0)),
            pl.BlockSpec((1, KP * PS, L), lambda b, p: (b, p, 0)),
        ],
        out_specs=pl.BlockSpec((1, 1, L), lambda b, p: (b, 0, 0)),
        out_shape=jax.ShapeDtypeStruct((B, 1, L), jnp.float32),
        scratch_shapes=[
            pltpu.VMEM((NP, L), jnp.float32),
            pltpu.VMEM((NP, KP * PS), jnp.int32),
        ],
        compiler_params=pltpu.CompilerParams(
            dimension_semantics=("parallel", "arbitrary"),
        ),
    )(x, attr)
    return out.reshape(B, L)
